# baseline (speedup 1.0000x reference)
import functools
import math

import jax
import jax.numpy as jnp
import numpy as np
from jax import lax
from jax.experimental import pallas as pl
from jax.experimental.pallas import tpu as pltpu
from jax.experimental.pallas import tpu_sc as plsc

_NUM_PROJS = 7
_BLOCK = 256
_SAMPLES = 256
_F32_MIN = float(np.finfo(np.float32).min)

_SC_LANES = 16
_CHUNK = 128
_D2 = 128


def _rank_body(codes_ref, p_ref, cnt_ref, *, n, nbins, chunk):
    nchunks = n // chunk
    bin_row = lax.broadcasted_iota(jnp.int32, (nbins, chunk), 0)
    ir = lax.broadcasted_iota(jnp.int32, (chunk, chunk), 0)
    ic = lax.broadcasted_iota(jnp.int32, (chunk, chunk), 1)
    triu_incl = (ir <= ic).astype(jnp.float32)

    dot = functools.partial(
        lax.dot_general, preferred_element_type=jnp.float32,
        precision=lax.Precision.HIGHEST)

    def onehot_at(c):
        codes_c = codes_ref[0, 0, c * chunk:(c + 1) * chunk]
        return (codes_c[None, :] == bin_row).astype(jnp.float32)

    for c in range(nchunks):
        cnt_ref[:, c:c + 1] = jnp.sum(onehot_at(c), axis=1, keepdims=True)

    counts = cnt_ref[:]
    ich_r = lax.broadcasted_iota(jnp.int32, (nchunks, nchunks), 0)
    ich_c = lax.broadcasted_iota(jnp.int32, (nchunks, nchunks), 1)
    chunk_tril = (ich_r < ich_c).astype(jnp.float32)
    carries = dot(counts, chunk_tril,
                  dimension_numbers=(((1,), (0,)), ((), ())))

    totals = jnp.sum(counts, axis=1, keepdims=True)
    ib_r = lax.broadcasted_iota(jnp.int32, (nbins, nbins), 0)
    ib_c = lax.broadcasted_iota(jnp.int32, (nbins, nbins), 1)
    tril_strict = (ib_r > ib_c).astype(jnp.float32)
    offsets = dot(tril_strict, totals,
                  dimension_numbers=(((1,), (0,)), ((), ())))
    obase = offsets + carries

    for c in range(nchunks):
        onehot = onehot_at(c)
        s_c = dot(onehot, triu_incl,
                  dimension_numbers=(((1,), (0,)), ((), ())))
        s_c = s_c + obase[:, c:c + 1]
        pos = jnp.sum(s_c * onehot, axis=0, keepdims=True) - 1.0
        p_ref[0, :, c * chunk:(c + 1) * chunk] = pos.astype(jnp.int32)


_SC_PARAMS = pltpu.CompilerParams(needs_layout_passes=False)


def _load_positions(p_hbm, wid, p_v):
    pltpu.sync_copy(p_hbm.at[wid], p_v)


def _make_sort_gather(bh, n, nsamp, bh0, nbh):
    mesh = plsc.VectorSubcoreMesh(core_axis_name="c", subcore_axis_name="s")
    f32 = jnp.float32
    nrow = n // _CHUNK

    @functools.partial(
        pl.kernel, mesh=mesh,
        out_type=[
            jax.ShapeDtypeStruct((nbh * n, _D2), f32),
            jax.ShapeDtypeStruct((nbh * n, _D2), f32),
            jax.ShapeDtypeStruct((nbh * nsamp, _D2), f32),
        ],
        scratch_types=[
            pltpu.VMEM((nrow, _CHUNK), jnp.int32),
            pltpu.VMEM((nrow, _CHUNK), jnp.int32),
            pltpu.VMEM((n,), jnp.int32),
            pltpu.VMEM((n,), jnp.int32),
            pltpu.VMEM((nsamp // _CHUNK, _CHUNK), jnp.int32),
            pltpu.VMEM((nsamp,), jnp.int32),
            pltpu.VMEM((_CHUNK, _D2), f32),
            pltpu.VMEM((_CHUNK, _D2), f32),
            pltpu.VMEM((_CHUNK, _D2), f32),
            pltpu.VMEM((_CHUNK, _D2), f32),
            pltpu.SemaphoreType.DMA,
            pltpu.SemaphoreType.DMA,
            pltpu.SemaphoreType.DMA,
            pltpu.SemaphoreType.DMA,
        ],
        compiler_params=_SC_PARAMS,
    )
    def sort_gather(pq_hbm, pk_hbm, samp_hbm, tq_hbm, tkv_hbm,
                    qs_hbm, kvs_hbm, kvsub_hbm,
                    pq_v, pk_v, invq_v, invk_v, samp_v, sampg_v,
                    bufqa, bufqb, bufka, bufkb, semqa, semqb, semka, semkb):
        wid = lax.axis_index("s") * 2 + lax.axis_index("c")
        lbh = wid // 2
        half = wid % 2
        gbh = bh0 + lbh
        base = gbh * n
        lbase = lbh * n
        _load_positions(pq_hbm, gbh, pq_v)
        _load_positions(pk_hbm, gbh, pk_v)
        pltpu.sync_copy(samp_hbm.at[gbh], samp_v)

        lane = lax.iota(jnp.int32, _SC_LANES)
        per_row = _CHUNK // _SC_LANES

        def inv_body(g, carry):
            r = g // per_row
            sl = pl.ds((g % per_row) * _SC_LANES, _SC_LANES)
            rid = lane + (g * _SC_LANES + base)
            plsc.store_scatter(invq_v, [pq_v[r, sl]], rid)
            plsc.store_scatter(invk_v, [pk_v[r, sl]], rid)
            return carry

        lax.fori_loop(0, n // _SC_LANES, inv_body, 0)

        def samp_body(g, carry):
            r = g // per_row
            sl = pl.ds((g % per_row) * _SC_LANES, _SC_LANES)
            sampg_v[pl.ds(g * _SC_LANES, _SC_LANES)] = plsc.load_gather(
                invk_v, [samp_v[r, sl]])
            return carry

        lax.fori_loop(0, nsamp // _SC_LANES, samp_body, 0)

        def gq(c, buf, sem):
            return pltpu.async_copy(
                tq_hbm.at[invq_v.at[pl.ds(c * _CHUNK, _CHUNK)]], buf, sem)

        def gkv(c, buf, sem):
            return pltpu.async_copy(
                tkv_hbm.at[invk_v.at[pl.ds(c * _CHUNK, _CHUNK)]], buf, sem)

        def wq(c, buf):
            pltpu.sync_copy(buf, qs_hbm.at[pl.ds(lbase + c * _CHUNK, _CHUNK)])

        def wkv(c, buf):
            pltpu.sync_copy(buf, kvs_hbm.at[pl.ds(lbase + c * _CHUNK, _CHUNK)])

        nhalf = n // (4 * _CHUNK)
        c_first = half * (nrow // 2)
        gq(c_first, bufqa, semqa)
        gkv(c_first, bufka, semka)

        def g_body(i, carry):
            c0 = c_first + 2 * i
            gq(c0 + 1, bufqb, semqb)
            gkv(c0 + 1, bufkb, semkb)
            pltpu.make_async_copy(
                tq_hbm.at[invq_v.at[pl.ds(c0 * _CHUNK, _CHUNK)]],
                bufqa, semqa).wait()
            wq(c0, bufqa)
            pltpu.make_async_copy(
                tkv_hbm.at[invk_v.at[pl.ds(c0 * _CHUNK, _CHUNK)]],
                bufka, semka).wait()
            wkv(c0, bufka)

            @pl.when(i < nhalf - 1)
            def _():
                gq(c0 + 2, bufqa, semqa)
                gkv(c0 + 2, bufka, semka)

            pltpu.make_async_copy(
                tq_hbm.at[invq_v.at[pl.ds((c0 + 1) * _CHUNK, _CHUNK)]],
                bufqb, semqb).wait()
            wq(c0 + 1, bufqb)
            pltpu.make_async_copy(
                tkv_hbm.at[invk_v.at[pl.ds((c0 + 1) * _CHUNK, _CHUNK)]],
                bufkb, semkb).wait()
            wkv(c0 + 1, bufkb)
            return carry

        lax.fori_loop(0, nhalf, g_body, 0)

        sls = pl.ds(half * _CHUNK, _CHUNK)
        pltpu.async_copy(tkv_hbm.at[sampg_v.at[sls]], bufqa, semqa).wait()
        pltpu.sync_copy(
            bufqa, kvsub_hbm.at[pl.ds(lbh * nsamp + half * _CHUNK, _CHUNK)])

    return sort_gather


def _make_unsort(bh, n, bh0, nbh):
    mesh = plsc.VectorSubcoreMesh(core_axis_name="c", subcore_axis_name="s")
    nrow = n // _CHUNK

    @functools.partial(
        pl.kernel, mesh=mesh,
        out_type=jax.ShapeDtypeStruct((nbh * n, _D2), jnp.float32),
        scratch_types=[
            pltpu.VMEM((nrow, _CHUNK), jnp.int32),
            pltpu.VMEM((n,), jnp.int32),
            pltpu.VMEM((_CHUNK, _D2), jnp.float32),
            pltpu.VMEM((_CHUNK, _D2), jnp.float32),
            pltpu.SemaphoreType.DMA,
            pltpu.SemaphoreType.DMA,
        ],
        compiler_params=_SC_PARAMS,
    )
    def unsort(pq_hbm, attn_hbm, out_hbm, pq_v, posg_v, bufa, bufb,
               sema, semb):
        wid = lax.axis_index("s") * 2 + lax.axis_index("c")
        lbh = wid // 2
        half = wid % 2
        base = lbh * n
        _load_positions(pq_hbm, bh0 + lbh, pq_v)

        per_row = _CHUNK // _SC_LANES

        def add_body(g, carry):
            r = g // per_row
            sl = pl.ds((g % per_row) * _SC_LANES, _SC_LANES)
            posg_v[pl.ds(g * _SC_LANES, _SC_LANES)] = pq_v[r, sl] + base
            return carry

        lax.fori_loop(0, n // _SC_LANES, add_body, 0)

        def g(c, buf, sem):
            return pltpu.async_copy(
                attn_hbm.at[posg_v.at[pl.ds(c * _CHUNK, _CHUNK)]], buf, sem)

        def w(c, buf):
            pltpu.sync_copy(buf, out_hbm.at[pl.ds(base + c * _CHUNK, _CHUNK)])

        nhalf = n // (4 * _CHUNK)
        c_first = half * (nrow // 2)
        g(c_first, bufa, sema)

        def g_body(i, carry):
            c0 = c_first + 2 * i
            g(c0 + 1, bufb, semb)
            pltpu.make_async_copy(
                attn_hbm.at[posg_v.at[pl.ds(c0 * _CHUNK, _CHUNK)]],
                bufa, sema).wait()
            w(c0, bufa)

            @pl.when(i < nhalf - 1)
            def _():
                g(c0 + 2, bufa, sema)

            pltpu.make_async_copy(
                attn_hbm.at[posg_v.at[pl.ds((c0 + 1) * _CHUNK, _CHUNK)]],
                bufb, semb).wait()
            w(c0 + 1, bufb)
            return carry

        lax.fori_loop(0, nhalf, g_body, 0)

    return unsort


def _attn_body(qs_ref, kvs_ref, kvsub_ref, samp_ref, out_ref,
               *, scale, num_blocks):
    dot = functools.partial(lax.dot_general, preferred_element_type=jnp.float32)

    kvsub = kvsub_ref[0]
    samp = samp_ref[0, 0]

    def block(j):
        sl = pl.ds(j * _BLOCK, _BLOCK)
        q = qs_ref[0, sl, :]
        kv = kvs_ref[0, sl, :]

        qk1 = dot(q, kv, dimension_numbers=(((1,), (1,)), ((), ()))) * scale
        m1 = jnp.max(qk1, axis=-1)
        e1 = jnp.exp(qk1 - m1[:, None])
        s1 = jnp.sum(e1, axis=-1)
        a1 = dot(e1, kv, dimension_numbers=(((1,), (0,)), ((), ())))
        a1 = a1 / s1[:, None]
        lse1 = m1 + jnp.log(s1)

        bias = jnp.where(samp // _BLOCK == j, _F32_MIN, 0.0)
        qk2 = dot(q, kvsub, dimension_numbers=(((1,), (1,)), ((), ()))) * scale
        qk2 = qk2 + bias[None, :].astype(jnp.float32)
        m2 = jnp.max(qk2, axis=-1)
        e2 = jnp.exp(qk2 - m2[:, None])
        s2 = jnp.sum(e2, axis=-1)
        a2 = dot(e2, kvsub, dimension_numbers=(((1,), (0,)), ((), ())))
        a2 = a2 / s2[:, None]
        lse2 = m2 + jnp.log(s2) + math.log(float(num_blocks))

        c = 1.0 / (1.0 + jnp.exp(lse2 - lse1))
        out_ref[0, sl, :] = c[:, None] * a1 + (1.0 - c[:, None]) * a2

    for j in range(num_blocks):
        block(j)


def kernel(query, key, value, proj_dir):
    b, h, n, d = query.shape
    bh = b * h
    num_blocks = n // _BLOCK
    scale = d ** (-0.5)

    enc_vec = (2 ** jnp.arange(_NUM_PROJS)).reshape(1, 1, 1, _NUM_PROJS)

    def lsh_hash(mat):
        mask = jnp.matmul(mat, proj_dir) > 0
        bin_ids = (mask * enc_vec).sum(-1)
        return bin_ids ^ (bin_ids >> 1)

    codes = jnp.stack([lsh_hash(query), lsh_hash(key)])
    codes = codes.reshape(2 * bh, 1, n).astype(jnp.int32)

    nbins = 2 ** _NUM_PROJS
    positions = pl.pallas_call(
        functools.partial(_rank_body, n=n, nbins=nbins, chunk=128),
        grid=(2 * bh,),
        in_specs=[pl.BlockSpec((1, 1, n), lambda i: (i, 0, 0))],
        out_specs=pl.BlockSpec((1, 1, n), lambda i: (i, 0, 0)),
        out_shape=jax.ShapeDtypeStruct((2 * bh, 1, n), jnp.int32),
        scratch_shapes=[pltpu.VMEM((nbins, n // 128), jnp.float32)],
    )(codes)
    positions = positions.reshape(2, bh, n // _CHUNK, _CHUNK)
    pq3, pk3 = positions[0], positions[1]

    sampled_set = jax.random.randint(jax.random.key(42), (b, h, _SAMPLES), 0, n)
    samp3 = sampled_set.reshape(bh, _SAMPLES // _CHUNK, _CHUNK).astype(jnp.int32)

    q2 = query.reshape(bh * n, d)
    k2 = key.reshape(bh * n, d)
    v2 = value.reshape(bh * n, d)
    table_q = jnp.concatenate([q2, jnp.zeros_like(q2)], axis=1)
    table_kv = jnp.concatenate([k2, v2], axis=1)

    samp_all = sampled_set.reshape(bh, 1, _SAMPLES).astype(jnp.int32)

    nbh = bh // 2
    halves = []
    for bh0 in (0, nbh):
        sort_gather = _make_sort_gather(bh, n, _SAMPLES, bh0, nbh)
        qs_f, kvs_f, kvsub_f = sort_gather(pq3, pk3, samp3, table_q, table_kv)

        qs = qs_f.reshape(nbh, n, _D2)
        kvs = kvs_f.reshape(nbh, n, _D2)
        kvsub = kvsub_f.reshape(nbh, _SAMPLES, _D2)
        samp = lax.slice_in_dim(samp_all, bh0, bh0 + nbh, axis=0)

        attn_sorted = pl.pallas_call(
            functools.partial(_attn_body, scale=scale, num_blocks=num_blocks),
            grid=(nbh,),
            in_specs=[
                pl.BlockSpec((1, n, _D2), lambda i: (i, 0, 0)),
                pl.BlockSpec((1, n, _D2), lambda i: (i, 0, 0)),
                pl.BlockSpec((1, _SAMPLES, _D2), lambda i: (i, 0, 0)),
                pl.BlockSpec((1, 1, _SAMPLES), lambda i: (i, 0, 0)),
            ],
            out_specs=pl.BlockSpec((1, n, _D2), lambda i: (i, 0, 0)),
            out_shape=jax.ShapeDtypeStruct((nbh, n, _D2), jnp.float32),
        )(qs, kvs, kvsub, samp)

        unsort = _make_unsort(bh, n, bh0, nbh)
        out_pack = unsort(pq3, attn_sorted.reshape(nbh * n, _D2))
        halves.append(out_pack[:, d:].reshape(nbh, n, d))

    return jnp.concatenate(halves, axis=0).reshape(b, h, n, d)


# speedup vs baseline: 1.0138x; 1.0138x over previous
import functools
import math

import jax
import jax.numpy as jnp
import numpy as np
from jax import lax
from jax.experimental import pallas as pl
from jax.experimental.pallas import tpu as pltpu
from jax.experimental.pallas import tpu_sc as plsc

_NUM_PROJS = 7
_BLOCK = 256
_SAMPLES = 256
_F32_MIN = float(np.finfo(np.float32).min)

_SC_LANES = 16
_CHUNK = 128
_D2 = 128


def _rank_body(codes_ref, p_ref, cnt_ref, *, n, nbins, chunk):
    nchunks = n // chunk
    bin_row = lax.broadcasted_iota(jnp.int32, (nbins, chunk), 0)
    ir = lax.broadcasted_iota(jnp.int32, (chunk, chunk), 0)
    ic = lax.broadcasted_iota(jnp.int32, (chunk, chunk), 1)
    triu_incl = (ir <= ic).astype(jnp.float32)

    dot = functools.partial(
        lax.dot_general, preferred_element_type=jnp.float32,
        precision=lax.Precision.HIGHEST)

    def onehot_at(c):
        codes_c = codes_ref[0, 0, c * chunk:(c + 1) * chunk]
        return (codes_c[None, :] == bin_row).astype(jnp.float32)

    for c in range(nchunks):
        cnt_ref[:, c:c + 1] = jnp.sum(onehot_at(c), axis=1, keepdims=True)

    counts = cnt_ref[:]
    ich_r = lax.broadcasted_iota(jnp.int32, (nchunks, nchunks), 0)
    ich_c = lax.broadcasted_iota(jnp.int32, (nchunks, nchunks), 1)
    chunk_tril = (ich_r < ich_c).astype(jnp.float32)
    carries = dot(counts, chunk_tril,
                  dimension_numbers=(((1,), (0,)), ((), ())))

    totals = jnp.sum(counts, axis=1, keepdims=True)
    ib_r = lax.broadcasted_iota(jnp.int32, (nbins, nbins), 0)
    ib_c = lax.broadcasted_iota(jnp.int32, (nbins, nbins), 1)
    tril_strict = (ib_r > ib_c).astype(jnp.float32)
    offsets = dot(tril_strict, totals,
                  dimension_numbers=(((1,), (0,)), ((), ())))
    obase = offsets + carries

    for c in range(nchunks):
        onehot = onehot_at(c)
        s_c = dot(onehot, triu_incl,
                  dimension_numbers=(((1,), (0,)), ((), ())))
        s_c = s_c + obase[:, c:c + 1]
        pos = jnp.sum(s_c * onehot, axis=0, keepdims=True) - 1.0
        p_ref[0, :, c * chunk:(c + 1) * chunk] = pos.astype(jnp.int32)


_SC_PARAMS = pltpu.CompilerParams(needs_layout_passes=False)


def _load_positions(p_hbm, wid, p_v):
    pltpu.sync_copy(p_hbm.at[wid], p_v)


def _make_sort_gather(bh, n, nsamp):
    mesh = plsc.VectorSubcoreMesh(core_axis_name="c", subcore_axis_name="s")
    f32 = jnp.float32
    nrow = n // _CHUNK

    @functools.partial(
        pl.kernel, mesh=mesh,
        out_type=[
            jax.ShapeDtypeStruct((bh * n, _D2), f32),
            jax.ShapeDtypeStruct((bh * n, _D2), f32),
            jax.ShapeDtypeStruct((bh * nsamp, _D2), f32),
        ],
        scratch_types=[
            pltpu.VMEM((nrow, _CHUNK), jnp.int32),
            pltpu.VMEM((nrow, _CHUNK), jnp.int32),
            pltpu.VMEM((n,), jnp.int32),
            pltpu.VMEM((n,), jnp.int32),
            pltpu.VMEM((nsamp // _CHUNK, _CHUNK), jnp.int32),
            pltpu.VMEM((nsamp,), jnp.int32),
            pltpu.VMEM((_CHUNK, _D2), f32),
            pltpu.VMEM((_CHUNK, _D2), f32),
            pltpu.VMEM((_CHUNK, _D2), f32),
            pltpu.VMEM((_CHUNK, _D2), f32),
            pltpu.SemaphoreType.DMA,
            pltpu.SemaphoreType.DMA,
            pltpu.SemaphoreType.DMA,
            pltpu.SemaphoreType.DMA,
        ],
        compiler_params=_SC_PARAMS,
    )
    def sort_gather(pq_hbm, pk_hbm, samp_hbm, tq_hbm, tkv_hbm,
                    qs_hbm, kvs_hbm, kvsub_hbm,
                    pq_v, pk_v, invq_v, invk_v, samp_v, sampg_v,
                    bufqa, bufqb, bufka, bufkb, semqa, semqb, semka, semkb):
        wid = lax.axis_index("s") * 2 + lax.axis_index("c")
        base = wid * n
        _load_positions(pq_hbm, wid, pq_v)
        _load_positions(pk_hbm, wid, pk_v)
        pltpu.sync_copy(samp_hbm.at[wid], samp_v)

        lane = lax.iota(jnp.int32, _SC_LANES)
        per_row = _CHUNK // _SC_LANES

        def inv_body(g, carry):
            r = g // per_row
            sl = pl.ds((g % per_row) * _SC_LANES, _SC_LANES)
            rid = lane + (g * _SC_LANES + base)
            plsc.store_scatter(invq_v, [pq_v[r, sl]], rid)
            plsc.store_scatter(invk_v, [pk_v[r, sl]], rid)
            return carry

        lax.fori_loop(0, n // _SC_LANES, inv_body, 0)

        def samp_body(g, carry):
            r = g // per_row
            sl = pl.ds((g % per_row) * _SC_LANES, _SC_LANES)
            sampg_v[pl.ds(g * _SC_LANES, _SC_LANES)] = plsc.load_gather(
                invk_v, [samp_v[r, sl]])
            return carry

        lax.fori_loop(0, nsamp // _SC_LANES, samp_body, 0)

        def gq(c, buf, sem):
            return pltpu.async_copy(
                tq_hbm.at[invq_v.at[pl.ds(c * _CHUNK, _CHUNK)]], buf, sem)

        def gkv(c, buf, sem):
            return pltpu.async_copy(
                tkv_hbm.at[invk_v.at[pl.ds(c * _CHUNK, _CHUNK)]], buf, sem)

        def wq(c, buf):
            pltpu.sync_copy(buf, qs_hbm.at[pl.ds(base + c * _CHUNK, _CHUNK)])

        def wkv(c, buf):
            pltpu.sync_copy(buf, kvs_hbm.at[pl.ds(base + c * _CHUNK, _CHUNK)])

        nhalf = n // (2 * _CHUNK)
        gq(0, bufqa, semqa)
        gkv(0, bufka, semka)

        def g_body(i, carry):
            c0 = 2 * i
            gq(c0 + 1, bufqb, semqb)
            gkv(c0 + 1, bufkb, semkb)
            pltpu.make_async_copy(
                tq_hbm.at[invq_v.at[pl.ds(c0 * _CHUNK, _CHUNK)]],
                bufqa, semqa).wait()
            wq(c0, bufqa)
            pltpu.make_async_copy(
                tkv_hbm.at[invk_v.at[pl.ds(c0 * _CHUNK, _CHUNK)]],
                bufka, semka).wait()
            wkv(c0, bufka)

            @pl.when(i < nhalf - 1)
            def _():
                gq(c0 + 2, bufqa, semqa)
                gkv(c0 + 2, bufka, semka)

            pltpu.make_async_copy(
                tq_hbm.at[invq_v.at[pl.ds((c0 + 1) * _CHUNK, _CHUNK)]],
                bufqb, semqb).wait()
            wq(c0 + 1, bufqb)
            pltpu.make_async_copy(
                tkv_hbm.at[invk_v.at[pl.ds((c0 + 1) * _CHUNK, _CHUNK)]],
                bufkb, semkb).wait()
            wkv(c0 + 1, bufkb)
            return carry

        lax.fori_loop(0, nhalf, g_body, 0)

        def s_body(c, carry):
            sl = pl.ds(c * _CHUNK, _CHUNK)
            pltpu.async_copy(tkv_hbm.at[sampg_v.at[sl]], bufqa, semqa).wait()
            pltpu.sync_copy(
                bufqa, kvsub_hbm.at[pl.ds(wid * nsamp + c * _CHUNK, _CHUNK)])
            return carry

        lax.fori_loop(0, nsamp // _CHUNK, s_body, 0)

    return sort_gather


def _make_unsort(bh, n):
    mesh = plsc.VectorSubcoreMesh(core_axis_name="c", subcore_axis_name="s")
    nrow = n // _CHUNK

    @functools.partial(
        pl.kernel, mesh=mesh,
        out_type=jax.ShapeDtypeStruct((bh * n, _D2), jnp.float32),
        scratch_types=[
            pltpu.VMEM((nrow, _CHUNK), jnp.int32),
            pltpu.VMEM((n,), jnp.int32),
            pltpu.VMEM((_CHUNK, _D2), jnp.float32),
            pltpu.VMEM((_CHUNK, _D2), jnp.float32),
            pltpu.SemaphoreType.DMA,
            pltpu.SemaphoreType.DMA,
        ],
        compiler_params=_SC_PARAMS,
    )
    def unsort(pq_hbm, attn_hbm, out_hbm, pq_v, posg_v, bufa, bufb,
               sema, semb):
        wid = lax.axis_index("s") * 2 + lax.axis_index("c")
        base = wid * n
        _load_positions(pq_hbm, wid, pq_v)

        per_row = _CHUNK // _SC_LANES

        def add_body(g, carry):
            r = g // per_row
            sl = pl.ds((g % per_row) * _SC_LANES, _SC_LANES)
            posg_v[pl.ds(g * _SC_LANES, _SC_LANES)] = pq_v[r, sl] + base
            return carry

        lax.fori_loop(0, n // _SC_LANES, add_body, 0)

        def g(c, buf, sem):
            return pltpu.async_copy(
                attn_hbm.at[posg_v.at[pl.ds(c * _CHUNK, _CHUNK)]], buf, sem)

        def w(c, buf):
            pltpu.sync_copy(buf, out_hbm.at[pl.ds(base + c * _CHUNK, _CHUNK)])

        nhalf = n // (2 * _CHUNK)
        g(0, bufa, sema)

        def g_body(i, carry):
            c0 = 2 * i
            g(c0 + 1, bufb, semb)
            pltpu.make_async_copy(
                attn_hbm.at[posg_v.at[pl.ds(c0 * _CHUNK, _CHUNK)]],
                bufa, sema).wait()
            w(c0, bufa)

            @pl.when(i < nhalf - 1)
            def _():
                g(c0 + 2, bufa, sema)

            pltpu.make_async_copy(
                attn_hbm.at[posg_v.at[pl.ds((c0 + 1) * _CHUNK, _CHUNK)]],
                bufb, semb).wait()
            w(c0 + 1, bufb)
            return carry

        lax.fori_loop(0, nhalf, g_body, 0)

    return unsort


def _attn_body(qs_ref, kvs_ref, kvsub_ref, samp_ref, out_ref,
               *, scale, num_blocks):
    dot = functools.partial(lax.dot_general, preferred_element_type=jnp.float32)

    kvsub = kvsub_ref[0]
    samp = samp_ref[0, 0]

    def block(j):
        sl = pl.ds(j * _BLOCK, _BLOCK)
        q = qs_ref[0, sl, :]
        kv = kvs_ref[0, sl, :]

        qk1 = dot(q, kv, dimension_numbers=(((1,), (1,)), ((), ()))) * scale
        m1 = jnp.max(qk1, axis=-1)
        e1 = jnp.exp(qk1 - m1[:, None])
        s1 = jnp.sum(e1, axis=-1)
        a1 = dot(e1, kv, dimension_numbers=(((1,), (0,)), ((), ())))
        a1 = a1 / s1[:, None]
        lse1 = m1 + jnp.log(s1)

        bias = jnp.where(samp // _BLOCK == j, _F32_MIN, 0.0)
        qk2 = dot(q, kvsub, dimension_numbers=(((1,), (1,)), ((), ()))) * scale
        qk2 = qk2 + bias[None, :].astype(jnp.float32)
        m2 = jnp.max(qk2, axis=-1)
        e2 = jnp.exp(qk2 - m2[:, None])
        s2 = jnp.sum(e2, axis=-1)
        a2 = dot(e2, kvsub, dimension_numbers=(((1,), (0,)), ((), ())))
        a2 = a2 / s2[:, None]
        lse2 = m2 + jnp.log(s2) + math.log(float(num_blocks))

        c = 1.0 / (1.0 + jnp.exp(lse2 - lse1))
        out_ref[0, sl, :] = c[:, None] * a1 + (1.0 - c[:, None]) * a2

    for j in range(num_blocks):
        block(j)


def kernel(query, key, value, proj_dir):
    b, h, n, d = query.shape
    bh = b * h
    num_blocks = n // _BLOCK
    scale = d ** (-0.5)

    enc_vec = (2 ** jnp.arange(_NUM_PROJS)).reshape(1, 1, 1, _NUM_PROJS)

    def lsh_hash(mat):
        mask = jnp.matmul(mat, proj_dir) > 0
        bin_ids = (mask * enc_vec).sum(-1)
        return bin_ids ^ (bin_ids >> 1)

    codes = jnp.stack([lsh_hash(query), lsh_hash(key)])
    codes = codes.reshape(2 * bh, 1, n).astype(jnp.int32)

    nbins = 2 ** _NUM_PROJS
    positions = pl.pallas_call(
        functools.partial(_rank_body, n=n, nbins=nbins, chunk=128),
        grid=(2 * bh,),
        in_specs=[pl.BlockSpec((1, 1, n), lambda i: (i, 0, 0))],
        out_specs=pl.BlockSpec((1, 1, n), lambda i: (i, 0, 0)),
        out_shape=jax.ShapeDtypeStruct((2 * bh, 1, n), jnp.int32),
        scratch_shapes=[pltpu.VMEM((nbins, n // 128), jnp.float32)],
    )(codes)
    positions = positions.reshape(2, bh, n // _CHUNK, _CHUNK)
    pq3, pk3 = positions[0], positions[1]

    sampled_set = jax.random.randint(jax.random.key(42), (b, h, _SAMPLES), 0, n)
    samp3 = sampled_set.reshape(bh, _SAMPLES // _CHUNK, _CHUNK).astype(jnp.int32)

    q2 = query.reshape(bh * n, d)
    k2 = key.reshape(bh * n, d)
    v2 = value.reshape(bh * n, d)
    table_q = jnp.concatenate([q2, jnp.zeros_like(q2)], axis=1)
    table_kv = jnp.concatenate([k2, v2], axis=1)

    sort_gather = _make_sort_gather(bh, n, _SAMPLES)
    qs_f, kvs_f, kvsub_f = sort_gather(pq3, pk3, samp3, table_q, table_kv)

    qs = qs_f.reshape(bh, n, _D2)
    kvs = kvs_f.reshape(bh, n, _D2)
    kvsub = kvsub_f.reshape(bh, _SAMPLES, _D2)
    samp = sampled_set.reshape(bh, 1, _SAMPLES).astype(jnp.int32)

    attn_sorted = pl.pallas_call(
        functools.partial(_attn_body, scale=scale, num_blocks=num_blocks),
        grid=(bh,),
        in_specs=[
            pl.BlockSpec((1, n, _D2), lambda i: (i, 0, 0)),
            pl.BlockSpec((1, n, _D2), lambda i: (i, 0, 0)),
            pl.BlockSpec((1, _SAMPLES, _D2), lambda i: (i, 0, 0)),
            pl.BlockSpec((1, 1, _SAMPLES), lambda i: (i, 0, 0)),
        ],
        out_specs=pl.BlockSpec((1, n, _D2), lambda i: (i, 0, 0)),
        out_shape=jax.ShapeDtypeStruct((bh, n, _D2), jnp.float32),
    )(qs, kvs, kvsub, samp)

    unsort = _make_unsort(bh, n)
    out_pack = unsort(pq3, attn_sorted.reshape(bh * n, _D2))
    return out_pack[:, d:].reshape(b, h, n, d)
